# baseline (speedup 1.0000x reference)
import functools

import jax
import jax.numpy as jnp
from jax import lax
from jax.experimental import pallas as pl
from jax.experimental.pallas import tpu as pltpu
from jax.experimental.pallas import tpu_sc as plsc

B = 4096
L = 2048
D = 32
V = 5

NC = 2
NS = 16
NW = NC * NS
RPW = B // NW

LANES = 16
SEG = 128
NSEG = L // SEG
DT = D // 8


def _comb_body(sem_ref, pos_ref, out_ref):
    sem = sem_ref[...]
    pos = pos_ref[...]
    out_ref[...] = sem[:, None, :] + pos[None, :, :]


def _build_comb(semantic_table, positional_table):
    comb3 = pl.pallas_call(
        _comb_body,
        out_shape=jax.ShapeDtypeStruct((V, L, D), jnp.float32),
    )(semantic_table, positional_table)
    return comb3.reshape(V * L, D)


_mesh = plsc.VectorSubcoreMesh(core_axis_name="c", subcore_axis_name="s")


@functools.partial(
    pl.kernel,
    out_type=jax.ShapeDtypeStruct((B, DT, NSEG, 8, SEG), jnp.float32),
    mesh=_mesh,
    scratch_types=[
        pltpu.VMEM((L,), jnp.int32),
        pltpu.VMEM((L,), jnp.int32),
        pltpu.VMEM((SEG, D), jnp.float32),
        pltpu.VMEM((SEG, D), jnp.float32),
        pltpu.VMEM((DT, 8, SEG), jnp.float32),
        pltpu.VMEM((DT, 8, SEG), jnp.float32),
        pltpu.VMEM_SHARED((V * L, D), jnp.float32),
        pltpu.SemaphoreType.DMA,
        pltpu.SemaphoreType.DMA,
        pltpu.SemaphoreType.DMA,
    ],
    compiler_params=pltpu.CompilerParams(
        use_tc_tiling_on_sc=False, needs_layout_passes=False),
)
def _sc_lookup(comb_hbm, x_hbm, out_hbm, xv, idxv, gb0, gb1, tb0, tb1,
               comb_sp, gsem, osem0, osem1):
    c = lax.axis_index("c")
    s = lax.axis_index("s")
    wid = s * NC + c
    base = wid * RPW
    iota = lax.broadcasted_iota(jnp.int32, (LANES,), 0)

    @pl.when(s == 0)
    def _load_comb():
        pltpu.sync_copy(comb_hbm, comb_sp)

    plsc.subcore_barrier()

    def fire_gather(seg, gb):
        pltpu.async_copy(
            comb_sp.at[idxv.at[pl.ds(seg * SEG, SEG)]], gb, gsem)

    def drain_gather(gb):
        pltpu.make_async_copy(
            comb_sp.at[idxv.at[pl.ds(0, SEG)]], gb, gsem).wait()

    def transpose(gb, tb):
        def group(g, carry):
            l16 = iota + g * LANES
            for d in range(D):
                vals = plsc.load_gather(
                    gb, [l16, jnp.full((LANES,), d, jnp.int32)])
                tb[d // 8, d % 8, pl.ds(g * LANES, LANES)] = vals
            return carry

        lax.fori_loop(0, SEG // LANES, group, 0, unroll=False)

    def row_body(r, carry):
        row = base + r
        pltpu.sync_copy(x_hbm.at[row], xv)

        def chunk(j, carry2):
            for k in range(SEG // LANES):
                off = j * SEG + k * LANES
                x16 = xv[pl.ds(off, LANES)]
                idxv[pl.ds(off, LANES)] = x16 * L + (iota + off)
            return carry2

        lax.fori_loop(0, NSEG, chunk, 0, unroll=False)

        fire_gather(0, gb0)

        def seg_pair(p, carry2):
            for par, gb, gnext, tb, osem in (
                (0, gb0, gb1, tb0, osem0),
                (1, gb1, gb0, tb1, osem1),
            ):
                seg = 2 * p + par
                drain_gather(gb)

                @pl.when(seg < NSEG - 1)
                def _fire_next():
                    fire_gather(seg + 1, gnext)

                @pl.when(jnp.logical_or(r > 0, p > 0))
                def _drain_out():
                    pltpu.make_async_copy(
                        tb, out_hbm.at[row, :, seg], osem).wait()

                transpose(gb, tb)
                pltpu.async_copy(tb, out_hbm.at[row, :, seg], osem)
            return carry2

        lax.fori_loop(0, NSEG // 2, seg_pair, 0, unroll=False)
        return carry

    lax.fori_loop(0, RPW, row_body, 0, unroll=False)
    pltpu.make_async_copy(tb0, out_hbm.at[base, :, 0], osem0).wait()
    pltpu.make_async_copy(tb1, out_hbm.at[base, :, 1], osem1).wait()


def kernel(x, semantic_table, positional_table):
    comb = _build_comb(semantic_table, positional_table)
    out5 = _sc_lookup(comb, x.astype(jnp.int32))
    return out5.transpose(0, 2, 4, 1, 3).reshape(B, L, D)


# speedup vs baseline: 1.7043x; 1.7043x over previous
import functools

import jax
import jax.numpy as jnp
from jax import lax
from jax.experimental import pallas as pl
from jax.experimental.pallas import tpu as pltpu
from jax.experimental.pallas import tpu_sc as plsc

B = 4096
L = 2048
D = 32
V = 5

NC = 2
NS = 16
NW = NC * NS
RPW = B // NW

LANES = 16
SEG = 128
NSEG = L // SEG
DT = D // 8


def _comb_body(sem_ref, pos_ref, out_ref):
    sem = sem_ref[...]
    pos = pos_ref[...]
    out_ref[...] = sem[:, None, :] + pos[None, :, :]


def _build_comb(semantic_table, positional_table):
    comb3 = pl.pallas_call(
        _comb_body,
        out_shape=jax.ShapeDtypeStruct((V, L, D), jnp.float32),
    )(semantic_table, positional_table)
    return comb3.reshape(V * L, D)


_mesh = plsc.VectorSubcoreMesh(core_axis_name="c", subcore_axis_name="s")


@functools.partial(
    pl.kernel,
    out_type=jax.ShapeDtypeStruct((B, DT, NSEG, 8, SEG), jnp.float32),
    mesh=_mesh,
    scratch_types=[
        pltpu.VMEM((L,), jnp.int32),
        pltpu.VMEM((L,), jnp.int32),
        pltpu.VMEM((SEG, D), jnp.float32),
        pltpu.VMEM((SEG, D), jnp.float32),
        pltpu.VMEM((DT, 8, SEG), jnp.float32),
        pltpu.VMEM((DT, 8, SEG), jnp.float32),
        pltpu.VMEM_SHARED((V * L, D), jnp.float32),
        pltpu.SemaphoreType.DMA,
        pltpu.SemaphoreType.DMA,
        pltpu.SemaphoreType.DMA,
    ],
    compiler_params=pltpu.CompilerParams(
        use_tc_tiling_on_sc=False, needs_layout_passes=False),
)
def _sc_lookup(comb_hbm, x_hbm, out_hbm, xv, idxv, gb0, gb1, tb0, tb1,
               comb_sp, gsem, osem0, osem1):
    c = lax.axis_index("c")
    s = lax.axis_index("s")
    wid = s * NC + c
    base = wid * RPW
    iota = lax.broadcasted_iota(jnp.int32, (LANES,), 0)

    @pl.when(s == 0)
    def _load_comb():
        pltpu.sync_copy(comb_hbm, comb_sp)

    plsc.subcore_barrier()

    def fire_gather(seg, gb):
        pltpu.async_copy(
            comb_sp.at[idxv.at[pl.ds(seg * SEG, SEG)]], gb, gsem)

    def drain_gather(gb):
        pltpu.make_async_copy(
            comb_sp.at[idxv.at[pl.ds(0, SEG)]], gb, gsem).wait()

    def transpose(gb, tb):
        def group(g, carry):
            l16 = iota + g * LANES
            vals = [
                plsc.load_gather(gb, [l16, jnp.full((LANES,), d, jnp.int32)])
                for d in range(D)
            ]
            for d in range(D):
                tb[d // 8, d % 8, pl.ds(g * LANES, LANES)] = vals[d]
            return carry

        lax.fori_loop(0, SEG // LANES, group, 0, unroll=False)

    def row_body(r, carry):
        row = base + r
        pltpu.sync_copy(x_hbm.at[row], xv)

        def chunk(j, carry2):
            for k in range(SEG // LANES):
                off = j * SEG + k * LANES
                x16 = xv[pl.ds(off, LANES)]
                idxv[pl.ds(off, LANES)] = x16 * L + (iota + off)
            return carry2

        lax.fori_loop(0, NSEG, chunk, 0, unroll=False)

        fire_gather(0, gb0)

        def seg_pair(p, carry2):
            for par, gb, gnext, tb, osem in (
                (0, gb0, gb1, tb0, osem0),
                (1, gb1, gb0, tb1, osem1),
            ):
                seg = 2 * p + par
                drain_gather(gb)

                @pl.when(seg < NSEG - 1)
                def _fire_next():
                    fire_gather(seg + 1, gnext)

                @pl.when(jnp.logical_or(r > 0, p > 0))
                def _drain_out():
                    pltpu.make_async_copy(
                        tb, out_hbm.at[row, :, seg], osem).wait()

                transpose(gb, tb)
                pltpu.async_copy(tb, out_hbm.at[row, :, seg], osem)
            return carry2

        lax.fori_loop(0, NSEG // 2, seg_pair, 0, unroll=False)
        return carry

    lax.fori_loop(0, RPW, row_body, 0, unroll=False)
    pltpu.make_async_copy(tb0, out_hbm.at[base, :, 0], osem0).wait()
    pltpu.make_async_copy(tb1, out_hbm.at[base, :, 1], osem1).wait()


def kernel(x, semantic_table, positional_table):
    comb = _build_comb(semantic_table, positional_table)
    out5 = _sc_lookup(comb, x.astype(jnp.int32))
    return out5.transpose(0, 2, 4, 1, 3).reshape(B, L, D)


# speedup vs baseline: 6.0274x; 3.5366x over previous
import functools

import jax
import jax.numpy as jnp
from jax import lax
from jax.experimental import pallas as pl
from jax.experimental.pallas import tpu as pltpu
from jax.experimental.pallas import tpu_sc as plsc

B = 4096
L = 2048
D = 32
V = 5
VP = 8

NC = 2
NS = 16
NW = NC * NS
RPW = B // NW

LANES = 16
SEG = 128
NSEG = L // SEG
DT = D // 8

_mesh = plsc.VectorSubcoreMesh(core_axis_name="c", subcore_axis_name="s")


@functools.partial(
    pl.kernel,
    out_type=jax.ShapeDtypeStruct((B, DT, NSEG, 8, SEG), jnp.float32),
    mesh=_mesh,
    scratch_types=[
        pltpu.VMEM((L,), jnp.int32),
        pltpu.VMEM((D, VP), jnp.float32),
        pltpu.VMEM((DT, 8, L), jnp.float32),
        pltpu.VMEM((DT, 8, SEG), jnp.float32),
        pltpu.VMEM((DT, 8, SEG), jnp.float32),
        pltpu.SemaphoreType.DMA,
        pltpu.SemaphoreType.DMA,
    ],
    compiler_params=pltpu.CompilerParams(
        use_tc_tiling_on_sc=False, needs_layout_passes=False),
)
def _sc_lookup(semT_hbm, posT_hbm, x_hbm, out_hbm, xv, semT, posT, tb0, tb1,
               osem0, osem1):
    c = lax.axis_index("c")
    s = lax.axis_index("s")
    wid = s * NC + c
    base = wid * RPW
    iota = lax.broadcasted_iota(jnp.int32, (LANES,), 0)

    pltpu.sync_copy(semT_hbm, semT)
    pltpu.sync_copy(posT_hbm, posT)

    def build_seg(seg, tb):
        def group(g, carry):
            off = seg * SEG + g * LANES
            x16 = xv[pl.ds(off, LANES)]
            vals = [
                plsc.load_gather(
                    semT, [jnp.full((LANES,), d, jnp.int32), x16])
                + posT[d // 8, d % 8, pl.ds(off, LANES)]
                for d in range(D)
            ]
            for d in range(D):
                tb[d // 8, d % 8, pl.ds(g * LANES, LANES)] = vals[d]
            return carry

        lax.fori_loop(0, SEG // LANES, group, 0, unroll=False)

    def row_body(r, carry):
        row = base + r
        pltpu.sync_copy(x_hbm.at[row], xv)

        def seg_pair(p, carry2):
            for par, tb, osem in ((0, tb0, osem0), (1, tb1, osem1)):
                seg = 2 * p + par

                @pl.when(jnp.logical_or(r > 0, p > 0))
                def _drain_out():
                    pltpu.make_async_copy(
                        tb, out_hbm.at[row, :, seg], osem).wait()

                build_seg(seg, tb)
                pltpu.async_copy(tb, out_hbm.at[row, :, seg], osem)
            return carry2

        lax.fori_loop(0, NSEG // 2, seg_pair, 0, unroll=False)
        return carry

    lax.fori_loop(0, RPW, row_body, 0, unroll=False)
    pltpu.make_async_copy(tb0, out_hbm.at[base, :, 0], osem0).wait()
    pltpu.make_async_copy(tb1, out_hbm.at[base, :, 1], osem1).wait()


def kernel(x, semantic_table, positional_table):
    semT = jnp.pad(semantic_table.T, ((0, 0), (0, VP - V)))
    posT = positional_table.T.reshape(DT, 8, L)
    out5 = _sc_lookup(semT, posT, x.astype(jnp.int32))
    return out5.transpose(0, 2, 4, 1, 3).reshape(B, L, D)
